# Initial kernel scaffold; baseline (speedup 1.0000x reference)
import jax
import jax.numpy as jnp
from jax.experimental import pallas as pl


def kernel(Q, grd0, grd1, grd2, grd3, batch_size, Wp0, bp0, gn_g0, gn_b0, Wp1, bp1, gn_g1, gn_b1, Wp2, bp2, gn_g2, gn_b2, Wp3, bp3, gn_g3, gn_b3, W_off, b_off, W_attn, b_attn, W_val, b_val, W_out, b_out, W1, b1, W2, b2, ln_g, ln_b):
    raise NotImplementedError("write your pallas kernel here")


# speedup vs baseline: 50.6209x; 50.6209x over previous
import functools

import jax
import jax.numpy as jnp
import numpy as np
from jax import lax
from jax.experimental import pallas as pl
from jax.experimental.pallas import tpu as pltpu
from jax.experimental.pallas import tpu_sc as plsc

EMBED = 256
HEADS = 8
LEVELS = 4
POINTS = 4
QDIM = 128
NQ = QDIM * QDIM
SHAPES = [(128, 128), (64, 64), (32, 32), (16, 16)]
CHANNELS = [24, 40, 112, 1280]
HID = 512
HD = EMBED // HEADS
NV = sum(h * w for h, w in SHAPES)
PAIRS = NQ * HEADS
EPP = LEVELS * POINTS * 4

BQ = 2048


def _level_body(g_ref, wp_ref, bp_ref, gg_ref, gb_ref, wval_ref, bval_ref,
                out_ref):
    G = g_ref[...]
    hw = G.shape[1]
    P = jnp.dot(wp_ref[...], G, preferred_element_type=jnp.float32)
    P = P + bp_ref[...][:, None]
    r = lax.broadcasted_iota(jnp.int32, (EMBED, 32), 0) // 8
    c = lax.broadcasted_iota(jnp.int32, (EMBED, 32), 1)
    M = (r == c).astype(jnp.float32)
    rowsum = jnp.sum(P, axis=1)[None, :]
    rowsq = jnp.sum(P * P, axis=1)[None, :]
    cnt = 8.0 * hw
    mu = jnp.dot(rowsum, M, preferred_element_type=jnp.float32) / cnt
    ex2 = jnp.dot(rowsq, M, preferred_element_type=jnp.float32) / cnt
    inv = lax.rsqrt(ex2 - mu * mu + 1e-5)
    mu_rows = jnp.dot(M, mu.T, preferred_element_type=jnp.float32)
    inv_rows = jnp.dot(M, inv.T, preferred_element_type=jnp.float32)
    Xn = (P - mu_rows) * inv_rows * gg_ref[...][:, None] + gb_ref[...][:, None]
    Y = lax.dot_general(Xn, wval_ref[...], (((0,), (1,)), ((), ())),
                        preferred_element_type=jnp.float32)
    out_ref[...] = Y + bval_ref[...][None, :]


def _level_value(g, wp, bp, gg, gb, wval, bval):
    c, hw = g.shape
    return pl.pallas_call(
        _level_body,
        out_shape=jax.ShapeDtypeStruct((hw, EMBED), jnp.float32),
    )(g, wp, bp, gg, gb, wval, bval)


def _qside_body(q_ref, wox_ref, woy_ref, box_ref, boy_ref, wa_ref, ba_ref,
                hcol_ref, loff_ref, wli_ref, hli_ref, wlf_ref, hlf_ref,
                i00, i10, i01, i11, w00, w10, w01, w11):
    qb = q_ref[...]
    dn = (((1,), (1,)), ((), ()))
    offx = lax.dot_general(qb, wox_ref[...], dn,
                           preferred_element_type=jnp.float32)
    offx = offx + box_ref[...][None, :]
    offy = lax.dot_general(qb, woy_ref[...], dn,
                           preferred_element_type=jnp.float32)
    offy = offy + boy_ref[...][None, :]
    logit = lax.dot_general(qb, wa_ref[...], dn,
                            preferred_element_type=jnp.float32)
    logit = logit + ba_ref[...][None, :]
    parts = []
    for h in range(HEADS):
        s = logit[:, h * 16:(h + 1) * 16]
        m = jnp.max(s, axis=1, keepdims=True)
        e = jnp.exp(s - m)
        parts.append(e / jnp.sum(e, axis=1, keepdims=True))
    aw = jnp.concatenate(parts, axis=1)

    qidx = pl.program_id(0) * BQ + lax.broadcasted_iota(jnp.int32, (BQ, 1), 0)
    refx = (qidx % QDIM).astype(jnp.float32) / (QDIM - 1.0)
    refy = (qidx // QDIM).astype(jnp.float32) / (QDIM - 1.0)
    wlf = wlf_ref[...][None, :]
    hlf = hlf_ref[...][None, :]
    wli = wli_ref[...][None, :]
    hli = hli_ref[...][None, :]
    loff = loff_ref[...][None, :]
    hcol = hcol_ref[...][None, :]
    sx = refx * wlf + offx - 0.5
    sy = refy * hlf + offy - 0.5
    x0f = jnp.floor(sx)
    y0f = jnp.floor(sy)
    wx1 = sx - x0f
    wx0 = 1.0 - wx1
    wy1 = sy - y0f
    wy0 = 1.0 - wy1
    x0 = x0f.astype(jnp.int32)
    y0 = y0f.astype(jnp.int32)

    outs_i = (i00, i10, i01, i11)
    outs_w = (w00, w10, w01, w11)
    for k, (cx, cy) in enumerate(((0, 0), (1, 0), (0, 1), (1, 1))):
        xs = x0 + cx
        ys = y0 + cy
        wgt = (wx1 if cx else wx0) * (wy1 if cy else wy0)
        valid = ((xs >= 0) & (xs <= wli - 1) & (ys >= 0) & (ys <= hli - 1))
        xc = jnp.clip(xs, 0, wli - 1)
        yc = jnp.clip(ys, 0, hli - 1)
        row = (loff + yc * wli + xc) * HEADS + hcol
        outs_i[k][...] = row
        outs_w[k][...] = jnp.where(valid, aw * wgt, 0.0)


def _qside(Q2, wox, woy, box, boy, wa, ba, cols):
    grid = (NQ // BQ,)
    qspec = pl.BlockSpec((BQ, EMBED), lambda i: (i, 0))
    full = lambda s: pl.BlockSpec(s, lambda i: tuple(0 for _ in s))
    cspec = full((128,))
    ospec = pl.BlockSpec((BQ, 128), lambda i: (i, 0))
    oshape = jax.ShapeDtypeStruct((NQ, 128), jnp.int32)
    wshape = jax.ShapeDtypeStruct((NQ, 128), jnp.float32)
    return pl.pallas_call(
        _qside_body,
        grid=grid,
        in_specs=[qspec, full((128, EMBED)), full((128, EMBED)), cspec, cspec,
                  full((128, EMBED)), cspec, cspec, cspec, cspec, cspec,
                  cspec, cspec],
        out_specs=[ospec] * 8,
        out_shape=[oshape] * 4 + [wshape] * 4,
    )(Q2, wox, woy, box, boy, wa, ba, *cols)


NW = 32
PAIRS_PER_W = PAIRS // NW
CHUNK_PAIRS = 16
N_CHUNKS = PAIRS_PER_W // CHUNK_PAIRS
CE = CHUNK_PAIRS * EPP
CROWS = CE // 128


def _sc_body(table_hbm, idx_hbm, w_hbm, out_hbm, idx_v, w_v, rows_v, out_v,
             sem):
    wid = lax.axis_index("s") * 2 + lax.axis_index("c")

    def chunk_body(ci, carry):
        pair0 = pl.multiple_of(wid * PAIRS_PER_W + ci * CHUNK_PAIRS,
                               CHUNK_PAIRS)
        e0 = pl.multiple_of(pair0 * EPP, CE)
        pltpu.sync_copy(idx_hbm.at[pl.ds(pl.multiple_of(e0 // 128, CROWS),
                                         CROWS)], idx_v)
        pltpu.sync_copy(w_hbm.at[pl.ds(e0, CE)], w_v)
        cps = []
        for j in range(CROWS):
            cps.append(pltpu.async_copy(
                table_hbm.at[idx_v.at[j]],
                rows_v.at[pl.ds(j * 128, 128)], sem))
        for cp in cps:
            cp.wait()

        def pair_body(p, carry2):
            def grp_body(g, accs):
                k0 = pl.multiple_of((p * EPP + g * 16), 16)
                wgrp = w_v[pl.ds(k0, 16)]

                def e_body(t, accs2):
                    a0, a1 = accs2
                    k = k0 + t
                    wv = lax.gather(
                        wgrp, jnp.full((16, 1), t, jnp.int32),
                        lax.GatherDimensionNumbers(
                            offset_dims=(), collapsed_slice_dims=(0,),
                            start_index_map=(0,)),
                        (1,), mode=lax.GatherScatterMode.PROMISE_IN_BOUNDS)
                    r0 = rows_v[k, pl.ds(0, 16)]
                    r1 = rows_v[k, pl.ds(16, 16)]
                    return (a0 + wv * r0, a1 + wv * r1)

                return lax.fori_loop(0, 16, e_body, accs, unroll=8)

            z = jnp.zeros((16,), jnp.float32)
            a0, a1 = lax.fori_loop(0, EPP // 16, grp_body, (z, z))
            out_v[p, pl.ds(0, 16)] = a0
            out_v[p, pl.ds(16, 16)] = a1
            return carry2

        lax.fori_loop(0, CHUNK_PAIRS, pair_body, 0)
        pltpu.sync_copy(out_v, out_hbm.at[pl.ds(pair0, CHUNK_PAIRS)])
        return carry

    lax.fori_loop(0, N_CHUNKS, chunk_body, 0)


def _sc_gather(table, idx2d, wflat):
    mesh = plsc.VectorSubcoreMesh(core_axis_name="c", subcore_axis_name="s")
    f = pl.kernel(
        _sc_body,
        mesh=mesh,
        compiler_params=pltpu.CompilerParams(use_tc_tiling_on_sc=False),
        out_type=jax.ShapeDtypeStruct((PAIRS, HD), jnp.float32),
        scratch_types=[
            pltpu.VMEM((CROWS, 128), jnp.int32),
            pltpu.VMEM((CE,), jnp.float32),
            pltpu.VMEM((CE, HD), jnp.float32),
            pltpu.VMEM((CHUNK_PAIRS, HD), jnp.float32),
            pltpu.SemaphoreType.DMA,
        ],
    )
    return f(table, idx2d, wflat)


def _ffn_body(a_ref, q_ref, wo_ref, bo_ref, w1_ref, b1_ref, w2_ref, b2_ref,
              g_ref, bb_ref, o_ref):
    dn = (((1,), (1,)), ((), ()))
    a = a_ref[...]
    q = q_ref[...]
    fused = lax.dot_general(a, wo_ref[...], dn,
                            preferred_element_type=jnp.float32)
    fused = fused + bo_ref[...][None, :] + 2.0 * q
    h1 = lax.dot_general(fused, w1_ref[...], dn,
                         preferred_element_type=jnp.float32)
    h1 = jnp.maximum(h1 + b1_ref[...][None, :], 0.0)
    h2 = lax.dot_general(h1, w2_ref[...], dn,
                         preferred_element_type=jnp.float32)
    h2 = h2 + b2_ref[...][None, :]
    mu = jnp.mean(h2, axis=1, keepdims=True)
    var = jnp.mean((h2 - mu) ** 2, axis=1, keepdims=True)
    o_ref[...] = fused + ((h2 - mu) * lax.rsqrt(var + 1e-5)) * \
        g_ref[...][None, :] + bb_ref[...][None, :]


def _ffn(attn_raw, Q2, wo, bo, w1, b1, w2, b2, g, b):
    grid = (NQ // BQ,)
    blk = pl.BlockSpec((BQ, EMBED), lambda i: (i, 0))
    full = lambda s: pl.BlockSpec(s, lambda i: tuple(0 for _ in s))
    return pl.pallas_call(
        _ffn_body,
        grid=grid,
        in_specs=[blk, blk, full((EMBED, EMBED)), full((EMBED,)),
                  full((HID, EMBED)), full((HID,)), full((EMBED, HID)),
                  full((EMBED,)), full((EMBED,)), full((EMBED,))],
        out_specs=blk,
        out_shape=jax.ShapeDtypeStruct((NQ, EMBED), jnp.float32),
    )(attn_raw, Q2, wo, bo, w1, b1, w2, b2, g, b)


def _col_consts():
    j = np.arange(HEADS * LEVELS * POINTS)
    lcol = (j % (LEVELS * POINTS)) // POINTS
    hcol = j // (LEVELS * POINTS)
    loff_tab = np.array([0, 16384, 20480, 21504], dtype=np.int32)
    wl = np.array([SHAPES[l][1] for l in range(LEVELS)], dtype=np.int32)
    hl = np.array([SHAPES[l][0] for l in range(LEVELS)], dtype=np.int32)
    return (
        jnp.asarray(hcol, dtype=jnp.int32),
        jnp.asarray(loff_tab[lcol], dtype=jnp.int32),
        jnp.asarray(wl[lcol], dtype=jnp.int32),
        jnp.asarray(hl[lcol], dtype=jnp.int32),
        jnp.asarray(wl[lcol], dtype=jnp.float32),
        jnp.asarray(hl[lcol], dtype=jnp.float32),
    )


def kernel(Q, grd0, grd1, grd2, grd3, batch_size, Wp0, bp0, gn_g0, gn_b0,
           Wp1, bp1, gn_g1, gn_b1, Wp2, bp2, gn_g2, gn_b2, Wp3, bp3, gn_g3,
           gn_b3, W_off, b_off, W_attn, b_attn, W_val, b_val, W_out, b_out,
           W1, b1, W2, b2, ln_g, ln_b):
    Q2 = Q.reshape(NQ, EMBED)
    grds = (grd0, grd1, grd2, grd3)
    projs = ((Wp0, bp0, gn_g0, gn_b0), (Wp1, bp1, gn_g1, gn_b1),
             (Wp2, bp2, gn_g2, gn_b2), (Wp3, bp3, gn_g3, gn_b3))
    vals = []
    for l in range(LEVELS):
        h, w = SHAPES[l]
        g = grds[l].reshape(CHANNELS[l], h * w)
        wp, bp, gg, gb = projs[l]
        vals.append(_level_value(g, wp, bp, gg, gb, W_val, b_val))
    table = jnp.concatenate(vals, axis=0).reshape(NV * HEADS, HD)

    cols = _col_consts()
    idx_parts = _qside(Q2, W_off[0::2], W_off[1::2], b_off[0::2], b_off[1::2],
                       W_attn, b_attn, cols)
    idx = jnp.concatenate(
        [a.reshape(PAIRS, LEVELS * POINTS) for a in idx_parts[:4]], axis=1)
    wts = jnp.concatenate(
        [a.reshape(PAIRS, LEVELS * POINTS) for a in idx_parts[4:]], axis=1)
    sc_out = _sc_gather(table, idx.reshape(PAIRS * EPP // 128, 128),
                        wts.reshape(PAIRS * EPP))
    attn_raw = sc_out.reshape(NQ, EMBED)
    final = _ffn(attn_raw, Q2, W_out, b_out, W1, b1, W2, b2, ln_g, ln_b)
    return final.reshape(1, NQ, EMBED)


# speedup vs baseline: 70.9649x; 1.4019x over previous
import functools

import jax
import jax.numpy as jnp
import numpy as np
from jax import lax
from jax.experimental import pallas as pl
from jax.experimental.pallas import tpu as pltpu
from jax.experimental.pallas import tpu_sc as plsc

EMBED = 256
HEADS = 8
LEVELS = 4
POINTS = 4
QDIM = 128
NQ = QDIM * QDIM
SHAPES = [(128, 128), (64, 64), (32, 32), (16, 16)]
CHANNELS = [24, 40, 112, 1280]
HID = 512
HD = EMBED // HEADS
NV = sum(h * w for h, w in SHAPES)
PAIRS = NQ * HEADS
EPP = LEVELS * POINTS * 4

BQ = 2048


def _level_body(g_ref, wp_ref, bp_ref, gg_ref, gb_ref, wval_ref, bval_ref,
                out_ref):
    G = g_ref[...]
    hw = G.shape[1]
    P = jnp.dot(wp_ref[...], G, preferred_element_type=jnp.float32)
    P = P + bp_ref[...][:, None]
    r = lax.broadcasted_iota(jnp.int32, (EMBED, 32), 0) // 8
    c = lax.broadcasted_iota(jnp.int32, (EMBED, 32), 1)
    M = (r == c).astype(jnp.float32)
    rowsum = jnp.sum(P, axis=1)[None, :]
    rowsq = jnp.sum(P * P, axis=1)[None, :]
    cnt = 8.0 * hw
    mu = jnp.dot(rowsum, M, preferred_element_type=jnp.float32) / cnt
    ex2 = jnp.dot(rowsq, M, preferred_element_type=jnp.float32) / cnt
    inv = lax.rsqrt(ex2 - mu * mu + 1e-5)
    mu_rows = jnp.dot(M, mu.T, preferred_element_type=jnp.float32)
    inv_rows = jnp.dot(M, inv.T, preferred_element_type=jnp.float32)
    Xn = (P - mu_rows) * inv_rows * gg_ref[...][:, None] + gb_ref[...][:, None]
    Y = lax.dot_general(Xn, wval_ref[...], (((0,), (1,)), ((), ())),
                        preferred_element_type=jnp.float32)
    out_ref[...] = Y + bval_ref[...][None, :]


def _level_value(g, wp, bp, gg, gb, wval, bval):
    c, hw = g.shape
    return pl.pallas_call(
        _level_body,
        out_shape=jax.ShapeDtypeStruct((hw, EMBED), jnp.float32),
    )(g, wp, bp, gg, gb, wval, bval)


def _qside_body(q_ref, wox_ref, woy_ref, box_ref, boy_ref, wa_ref, ba_ref,
                hcol_ref, loff_ref, wli_ref, hli_ref, wlf_ref, hlf_ref,
                i00, i10, i01, i11, w00, w10, w01, w11):
    qb = q_ref[...]
    dn = (((1,), (1,)), ((), ()))
    offx = lax.dot_general(qb, wox_ref[...], dn,
                           preferred_element_type=jnp.float32)
    offx = offx + box_ref[...][None, :]
    offy = lax.dot_general(qb, woy_ref[...], dn,
                           preferred_element_type=jnp.float32)
    offy = offy + boy_ref[...][None, :]
    logit = lax.dot_general(qb, wa_ref[...], dn,
                            preferred_element_type=jnp.float32)
    logit = logit + ba_ref[...][None, :]
    parts = []
    for h in range(HEADS):
        s = logit[:, h * 16:(h + 1) * 16]
        m = jnp.max(s, axis=1, keepdims=True)
        e = jnp.exp(s - m)
        parts.append(e / jnp.sum(e, axis=1, keepdims=True))
    aw = jnp.concatenate(parts, axis=1)

    qidx = pl.program_id(0) * BQ + lax.broadcasted_iota(jnp.int32, (BQ, 1), 0)
    refx = (qidx % QDIM).astype(jnp.float32) / (QDIM - 1.0)
    refy = (qidx // QDIM).astype(jnp.float32) / (QDIM - 1.0)
    wlf = wlf_ref[...][None, :]
    hlf = hlf_ref[...][None, :]
    wli = wli_ref[...][None, :]
    hli = hli_ref[...][None, :]
    loff = loff_ref[...][None, :]
    hcol = hcol_ref[...][None, :]
    sx = refx * wlf + offx - 0.5
    sy = refy * hlf + offy - 0.5
    x0f = jnp.floor(sx)
    y0f = jnp.floor(sy)
    wx1 = sx - x0f
    wx0 = 1.0 - wx1
    wy1 = sy - y0f
    wy0 = 1.0 - wy1
    x0 = x0f.astype(jnp.int32)
    y0 = y0f.astype(jnp.int32)

    outs_i = (i00, i10, i01, i11)
    outs_w = (w00, w10, w01, w11)
    for k, (cx, cy) in enumerate(((0, 0), (1, 0), (0, 1), (1, 1))):
        xs = x0 + cx
        ys = y0 + cy
        wgt = (wx1 if cx else wx0) * (wy1 if cy else wy0)
        valid = ((xs >= 0) & (xs <= wli - 1) & (ys >= 0) & (ys <= hli - 1))
        xc = jnp.clip(xs, 0, wli - 1)
        yc = jnp.clip(ys, 0, hli - 1)
        row = (loff + yc * wli + xc) * HEADS + hcol
        outs_i[k][...] = row
        outs_w[k][...] = jnp.where(valid, aw * wgt, 0.0)


def _qside(Q2, wox, woy, box, boy, wa, ba, cols):
    grid = (NQ // BQ,)
    qspec = pl.BlockSpec((BQ, EMBED), lambda i: (i, 0))
    full = lambda s: pl.BlockSpec(s, lambda i: tuple(0 for _ in s))
    cspec = full((128,))
    ospec = pl.BlockSpec((BQ, 128), lambda i: (i, 0))
    oshape = jax.ShapeDtypeStruct((NQ, 128), jnp.int32)
    wshape = jax.ShapeDtypeStruct((NQ, 128), jnp.float32)
    return pl.pallas_call(
        _qside_body,
        grid=grid,
        in_specs=[qspec, full((128, EMBED)), full((128, EMBED)), cspec, cspec,
                  full((128, EMBED)), cspec, cspec, cspec, cspec, cspec,
                  cspec, cspec],
        out_specs=[ospec] * 8,
        out_shape=[oshape] * 4 + [wshape] * 4,
    )(Q2, wox, woy, box, boy, wa, ba, *cols)


NW = 32
PAIRS_PER_W = PAIRS // NW
CHUNK_PAIRS = 16
N_CHUNKS = PAIRS_PER_W // CHUNK_PAIRS
CE = CHUNK_PAIRS * EPP
CROWS = CE // 128


def _sc_body(table_hbm, idx_hbm, w_hbm, out_hbm,
             idx_v0, idx_v1, w_v0, w_v1, rows_v0, rows_v1, out_v,
             sem0, sem1):
    wid = lax.axis_index("s") * 2 + lax.axis_index("c")
    idx_vs = (idx_v0, idx_v1)
    w_vs = (w_v0, w_v1)
    rows_vs = (rows_v0, rows_v1)
    sems = (sem0, sem1)

    def pair_base(ci):
        return pl.multiple_of(wid * PAIRS_PER_W + ci * CHUNK_PAIRS,
                              CHUNK_PAIRS)

    def stage(ci, b):
        e0 = pl.multiple_of(pair_base(ci) * EPP, CE)
        pltpu.sync_copy(idx_hbm.at[pl.ds(e0, CE)], idx_vs[b])
        pltpu.sync_copy(w_hbm.at[pl.ds(e0, CE)], w_vs[b])
        pltpu.async_copy(table_hbm.at[idx_vs[b]], rows_vs[b], sems[b])

    def wait_gathers(b):
        pltpu.make_async_copy(table_hbm.at[idx_vs[b]], rows_vs[b],
                              sems[b]).wait()

    def compute(ci, b):
        rows_v = rows_vs[b]
        w_v = w_vs[b]

        def splat(vec, t):
            return lax.gather(
                vec, jnp.full((16, 1), t, jnp.int32),
                lax.GatherDimensionNumbers(
                    offset_dims=(), collapsed_slice_dims=(0,),
                    start_index_map=(0,)),
                (1,), mode=lax.GatherScatterMode.PROMISE_IN_BOUNDS)

        def pair_body(p, carry2):
            def grp_body(g, accs):
                k0 = pl.multiple_of((p * EPP + g * 16), 16)
                wgrp = w_v[pl.ds(k0, 16)]

                def e_body(t, accs2):
                    a0, a1, b0, b1 = accs2
                    t0 = 2 * t
                    t1 = 2 * t + 1
                    wv0 = splat(wgrp, t0)
                    wv1 = splat(wgrp, t1)
                    r00 = rows_v[k0 + t0, pl.ds(0, 16)]
                    r01 = rows_v[k0 + t0, pl.ds(16, 16)]
                    r10 = rows_v[k0 + t1, pl.ds(0, 16)]
                    r11 = rows_v[k0 + t1, pl.ds(16, 16)]
                    return (a0 + wv0 * r00, a1 + wv0 * r01,
                            b0 + wv1 * r10, b1 + wv1 * r11)

                return lax.fori_loop(0, 8, e_body, accs, unroll=8)

            z = jnp.zeros((16,), jnp.float32)
            a0, a1, b0, b1 = lax.fori_loop(0, EPP // 16, grp_body,
                                           (z, z, z, z))
            out_v[p, pl.ds(0, 16)] = a0 + b0
            out_v[p, pl.ds(16, 16)] = a1 + b1
            return carry2

        lax.fori_loop(0, CHUNK_PAIRS, pair_body, 0)
        pltpu.sync_copy(out_v, out_hbm.at[pl.ds(pair_base(ci), CHUNK_PAIRS)])

    stage(0, 0)

    def outer(i, carry):
        for b in (0, 1):
            ci = 2 * i + b

            @pl.when(ci + 1 < N_CHUNKS)
            def _():
                stage(ci + 1, 1 - b)

            wait_gathers(b)
            compute(ci, b)
        return carry

    lax.fori_loop(0, N_CHUNKS // 2, outer, 0)


def _sc_gather(table, idx2d, wflat):
    mesh = plsc.VectorSubcoreMesh(core_axis_name="c", subcore_axis_name="s")
    f = pl.kernel(
        _sc_body,
        mesh=mesh,
        compiler_params=pltpu.CompilerParams(use_tc_tiling_on_sc=False),
        out_type=jax.ShapeDtypeStruct((PAIRS, HD), jnp.float32),
        scratch_types=[
            pltpu.VMEM((CE,), jnp.int32),
            pltpu.VMEM((CE,), jnp.int32),
            pltpu.VMEM((CE,), jnp.float32),
            pltpu.VMEM((CE,), jnp.float32),
            pltpu.VMEM((CE, HD), jnp.float32),
            pltpu.VMEM((CE, HD), jnp.float32),
            pltpu.VMEM((CHUNK_PAIRS, HD), jnp.float32),
            pltpu.SemaphoreType.DMA,
            pltpu.SemaphoreType.DMA,
        ],
    )
    return f(table, idx2d, wflat)


def _ffn_body(a_ref, q_ref, wo_ref, bo_ref, w1_ref, b1_ref, w2_ref, b2_ref,
              g_ref, bb_ref, o_ref):
    dn = (((1,), (1,)), ((), ()))
    a = a_ref[...]
    q = q_ref[...]
    fused = lax.dot_general(a, wo_ref[...], dn,
                            preferred_element_type=jnp.float32)
    fused = fused + bo_ref[...][None, :] + 2.0 * q
    h1 = lax.dot_general(fused, w1_ref[...], dn,
                         preferred_element_type=jnp.float32)
    h1 = jnp.maximum(h1 + b1_ref[...][None, :], 0.0)
    h2 = lax.dot_general(h1, w2_ref[...], dn,
                         preferred_element_type=jnp.float32)
    h2 = h2 + b2_ref[...][None, :]
    mu = jnp.mean(h2, axis=1, keepdims=True)
    var = jnp.mean((h2 - mu) ** 2, axis=1, keepdims=True)
    o_ref[...] = fused + ((h2 - mu) * lax.rsqrt(var + 1e-5)) * \
        g_ref[...][None, :] + bb_ref[...][None, :]


def _ffn(attn_raw, Q2, wo, bo, w1, b1, w2, b2, g, b):
    grid = (NQ // BQ,)
    blk = pl.BlockSpec((BQ, EMBED), lambda i: (i, 0))
    full = lambda s: pl.BlockSpec(s, lambda i: tuple(0 for _ in s))
    return pl.pallas_call(
        _ffn_body,
        grid=grid,
        in_specs=[blk, blk, full((EMBED, EMBED)), full((EMBED,)),
                  full((HID, EMBED)), full((HID,)), full((EMBED, HID)),
                  full((EMBED,)), full((EMBED,)), full((EMBED,))],
        out_specs=blk,
        out_shape=jax.ShapeDtypeStruct((NQ, EMBED), jnp.float32),
    )(attn_raw, Q2, wo, bo, w1, b1, w2, b2, g, b)


def _col_consts():
    j = np.arange(HEADS * LEVELS * POINTS)
    lcol = (j % (LEVELS * POINTS)) // POINTS
    hcol = j // (LEVELS * POINTS)
    loff_tab = np.array([0, 16384, 20480, 21504], dtype=np.int32)
    wl = np.array([SHAPES[l][1] for l in range(LEVELS)], dtype=np.int32)
    hl = np.array([SHAPES[l][0] for l in range(LEVELS)], dtype=np.int32)
    return (
        jnp.asarray(hcol, dtype=jnp.int32),
        jnp.asarray(loff_tab[lcol], dtype=jnp.int32),
        jnp.asarray(wl[lcol], dtype=jnp.int32),
        jnp.asarray(hl[lcol], dtype=jnp.int32),
        jnp.asarray(wl[lcol], dtype=jnp.float32),
        jnp.asarray(hl[lcol], dtype=jnp.float32),
    )


def kernel(Q, grd0, grd1, grd2, grd3, batch_size, Wp0, bp0, gn_g0, gn_b0,
           Wp1, bp1, gn_g1, gn_b1, Wp2, bp2, gn_g2, gn_b2, Wp3, bp3, gn_g3,
           gn_b3, W_off, b_off, W_attn, b_attn, W_val, b_val, W_out, b_out,
           W1, b1, W2, b2, ln_g, ln_b):
    Q2 = Q.reshape(NQ, EMBED)
    grds = (grd0, grd1, grd2, grd3)
    projs = ((Wp0, bp0, gn_g0, gn_b0), (Wp1, bp1, gn_g1, gn_b1),
             (Wp2, bp2, gn_g2, gn_b2), (Wp3, bp3, gn_g3, gn_b3))
    vals = []
    for l in range(LEVELS):
        h, w = SHAPES[l]
        g = grds[l].reshape(CHANNELS[l], h * w)
        wp, bp, gg, gb = projs[l]
        vals.append(_level_value(g, wp, bp, gg, gb, W_val, b_val))
    table = jnp.concatenate(vals, axis=0).reshape(NV * HEADS, HD)

    cols = _col_consts()
    idx_parts = _qside(Q2, W_off[0::2], W_off[1::2], b_off[0::2], b_off[1::2],
                       W_attn, b_attn, cols)
    idx = jnp.concatenate(
        [a.reshape(PAIRS, LEVELS * POINTS) for a in idx_parts[:4]], axis=1)
    wts = jnp.concatenate(
        [a.reshape(PAIRS, LEVELS * POINTS) for a in idx_parts[4:]], axis=1)
    sc_out = _sc_gather(table, idx.reshape(PAIRS * EPP),
                        wts.reshape(PAIRS * EPP))
    attn_raw = sc_out.reshape(NQ, EMBED)
    final = _ffn(attn_raw, Q2, W_out, b_out, W1, b1, W2, b2, ln_g, ln_b)
    return final.reshape(1, NQ, EMBED)
